# Initial kernel scaffold; baseline (speedup 1.0000x reference)
import jax
import jax.numpy as jnp
from jax.experimental import pallas as pl


def kernel(x, edge_index, batch, W1, b1, W2, b2, W3, b3, Wg, a_src, a_dst, bg, Wc1, bc1, Wc2, bc2):
    raise NotImplementedError("write your pallas kernel here")


# speedup vs baseline: 27.9814x; 27.9814x over previous
import functools

import jax
import jax.numpy as jnp
from jax import lax
from jax.experimental import pallas as pl
from jax.experimental.pallas import tpu as pltpu
from jax.experimental.pallas import tpu_sc as plsc

F32 = jnp.float32
I32 = jnp.int32

NN = 10000
EE = 320000
DD = 128
HID = 64
GG = 16
NH = 4
HD = 16

NC = 2
NS = 16
NW = NC * NS
CH = 128
CPW = -(-EE // (NW * CH))
EPAD = NW * CPW * CH
NACC = 10240
ZR = NACC // NS
WR = NN // NS

_MESH = plsc.VectorSubcoreMesh(core_axis_name="c", subcore_axis_name="s")
_SC_PARAMS = pltpu.CompilerParams(use_tc_tiling_on_sc=False)
_PREC = lax.Precision.HIGHEST
_NGRID = 5
_NB = NN // _NGRID


def _seg_scatter(mode, feat, table, src1d, dst1d):
  zeros = jnp.zeros((NACC, feat), F32)

  scratch = [
      pltpu.VMEM((CH,), I32),
      pltpu.VMEM((CH,), I32),
      pltpu.VMEM((CH, feat), F32),
      pltpu.VMEM_SHARED((NACC, feat), F32),
  ]

  @functools.partial(
      pl.kernel,
      out_type=jax.ShapeDtypeStruct((NC, NACC, feat), F32),
      mesh=_MESH,
      scratch_types=scratch,
      compiler_params=_SC_PARAMS,
      name=f"sc_seg_scatter_{mode}_{feat}",
  )
  def run(table_hbm, src_hbm, dst_hbm, zeros_hbm, out_hbm, sidx, didx, rows,
          acc):
    c = lax.axis_index("c")
    s = lax.axis_index("s")
    w = c * NS + s

    pltpu.sync_copy(zeros_hbm.at[pl.ds(s * ZR, ZR)],
                    acc.at[pl.ds(s * ZR, ZR)])
    if mode == "const":
      for k in range(CH):
        rows[k, :] = jnp.full((HD,), 1.0, F32)
    plsc.subcore_barrier()

    @pl.loop(0, CPW)
    def _(j):
      r = w * CPW + j
      pltpu.sync_copy(dst_hbm.at[pl.ds(r * CH, CH)], didx)
      if mode == "gather":
        pltpu.sync_copy(src_hbm.at[pl.ds(r * CH, CH)], sidx)
        pltpu.sync_copy(table_hbm.at[sidx], rows)
      elif mode == "linear":
        pltpu.sync_copy(table_hbm.at[pl.ds(r * CH, CH)], rows)
      pltpu.sync_copy(rows, acc.at[didx], add=True)

    plsc.subcore_barrier()
    pltpu.sync_copy(acc.at[pl.ds(s * ZR, ZR)],
                    out_hbm.at[c, pl.ds(s * ZR, ZR)])

  return run(table, src1d, dst1d, zeros)


def _gat_gather(tsrc, tald, src1d, dst1d):
  scratch = [
      pltpu.VMEM((CH,), I32),
      pltpu.VMEM((CH,), I32),
      pltpu.VMEM((CH, 80), F32),
      pltpu.VMEM((CH, 16), F32),
  ]

  @functools.partial(
      pl.kernel,
      out_type=(jax.ShapeDtypeStruct((EPAD, 80), F32),
                jax.ShapeDtypeStruct((EPAD, 16), F32)),
      mesh=_MESH,
      scratch_types=scratch,
      compiler_params=_SC_PARAMS,
      name="sc_gat_gather",
  )
  def run(tsrc_hbm, tald_hbm, src_hbm, dst_hbm, gsrc_hbm, gz_hbm, sidx, didx,
          rows80, rows16):
    c = lax.axis_index("c")
    s = lax.axis_index("s")
    w = c * NS + s

    @pl.loop(0, CPW)
    def _(j):
      r = w * CPW + j
      pltpu.sync_copy(src_hbm.at[pl.ds(r * CH, CH)], sidx)
      pltpu.sync_copy(dst_hbm.at[pl.ds(r * CH, CH)], didx)
      pltpu.sync_copy(tsrc_hbm.at[sidx], rows80)
      pltpu.sync_copy(tald_hbm.at[didx], rows16)
      pltpu.sync_copy(rows80, gsrc_hbm.at[pl.ds(r * CH, CH)])
      pltpu.sync_copy(rows16, gz_hbm.at[pl.ds(r * CH, CH)])

  return run(tsrc, tald, src1d, dst1d)


def _tc1(x, c0, c1, w1):
  def body(x_ref, c0_ref, c1_ref, w_ref, t_ref, dinv_ref):
    deg = 1.0 + c0_ref[:, 0:1] + c1_ref[:, 0:1]
    dinv = lax.rsqrt(deg)
    h = jnp.dot(x_ref[...], w_ref[...], precision=_PREC,
                preferred_element_type=F32)
    t_ref[...] = dinv * h
    dinv_ref[...] = dinv

  return pl.pallas_call(
      body,
      grid=(_NGRID,),
      in_specs=[pl.BlockSpec((_NB, DD), lambda i: (i, 0)),
                pl.BlockSpec((_NB, HD), lambda i: (i, 0)),
                pl.BlockSpec((_NB, HD), lambda i: (i, 0)),
                pl.BlockSpec((DD, HID), lambda i: (0, 0))],
      out_specs=(pl.BlockSpec((_NB, HID), lambda i: (i, 0)),
                 pl.BlockSpec((_NB, 1), lambda i: (i, 0))),
      out_shape=(jax.ShapeDtypeStruct((NN, HID), F32),
                 jax.ShapeDtypeStruct((NN, 1), F32)),
  )(x, c0, c1, w1)


def _tc_layer(p0, p1, tprev, dinv, b, wnext, res, fin, fout):
  def body(*refs):
    if res is None:
      p0_ref, p1_ref, t_ref, d_ref, b_ref, w_ref, y_ref, tn_ref = refs
    else:
      p0_ref, p1_ref, t_ref, d_ref, b_ref, w_ref, r_ref, y_ref, tn_ref = refs
    dinv_v = d_ref[...]
    y = jnp.maximum(dinv_v * (p0_ref[...] + p1_ref[...] + t_ref[...])
                    + b_ref[...], 0.0)
    if res is not None:
      y = y + r_ref[...]
    y_ref[...] = y
    tn_ref[...] = dinv_v * jnp.dot(y, w_ref[...], precision=_PREC,
                                   preferred_element_type=F32)

  args = [p0, p1, tprev, dinv, b, wnext] + ([] if res is None else [res])
  nspec = lambda f: pl.BlockSpec((_NB, f), lambda i: (i, 0))
  wspec = lambda a, bb: pl.BlockSpec((a, bb), lambda i: (0, 0))
  in_specs = [nspec(fin), nspec(fin), nspec(fin), nspec(1), wspec(1, fin),
              wspec(fin, fout)] + ([] if res is None else [nspec(fin)])
  return pl.pallas_call(
      body,
      grid=(_NGRID,),
      in_specs=in_specs,
      out_specs=(nspec(fin), nspec(fout)),
      out_shape=(jax.ShapeDtypeStruct((NN, fin), F32),
                 jax.ShapeDtypeStruct((NN, fout), F32)),
  )(*args)


def _tc_gatprep(p0, p1, t3, dinv, b3, wg, asrc, adst):
  def body(p0_ref, p1_ref, t_ref, d_ref, b_ref, wg_ref, as_ref, ad_ref,
           tsrc_ref, tald_ref):
    x3 = jnp.maximum(d_ref[...] * (p0_ref[...] + p1_ref[...] + t_ref[...])
                     + b_ref[...], 0.0)
    hh = jnp.dot(x3, wg_ref[...], precision=_PREC, preferred_element_type=F32)
    als = jnp.dot(hh, as_ref[...], precision=_PREC, preferred_element_type=F32)
    ald = jnp.dot(hh, ad_ref[...], precision=_PREC, preferred_element_type=F32)
    pad = jnp.zeros((_NB, 12), F32)
    tsrc_ref[...] = jnp.concatenate([hh, als, pad], axis=1)
    tald_ref[...] = jnp.concatenate([ald, pad], axis=1)

  nspec = lambda f: pl.BlockSpec((_NB, f), lambda i: (i, 0))
  wspec = lambda a, bb: pl.BlockSpec((a, bb), lambda i: (0, 0))
  return pl.pallas_call(
      body,
      grid=(_NGRID,),
      in_specs=[nspec(32), nspec(32), nspec(32), nspec(1), wspec(1, 32),
                wspec(32, 64), wspec(64, NH), wspec(64, NH)],
      out_specs=(nspec(80), nspec(16)),
      out_shape=(jax.ShapeDtypeStruct((NN, 80), F32),
                 jax.ShapeDtypeStruct((NN, 16), F32)),
  )(p0, p1, t3, dinv, b3, wg, asrc, adst)


_EGRID = 32
_EB = EPAD // _EGRID


def _tc_edge(gsrc, gz):
  def body(g_ref, z_ref, s_ref):
    g = g_ref[...]
    z = g[:, 64:80] + z_ref[...]
    w16 = jnp.exp(jnp.maximum(z, 0.2 * z))
    parts = [g[:, h * HD:(h + 1) * HD] * w16[:, h:h + 1] for h in range(NH)]
    s_ref[...] = jnp.concatenate(parts + [w16], axis=1)

  return pl.pallas_call(
      body,
      grid=(_EGRID,),
      in_specs=[pl.BlockSpec((_EB, 80), lambda i: (i, 0)),
                pl.BlockSpec((_EB, 16), lambda i: (i, 0))],
      out_specs=pl.BlockSpec((_EB, 80), lambda i: (i, 0)),
      out_shape=jax.ShapeDtypeStruct((EPAD, 80), F32),
  )(gsrc, gz)


def _tc_att(p0, p1, tsrc, tald, bg):
  def body(p0_ref, p1_ref, ts_ref, ta_ref, bg_ref, out_ref):
    agg = p0_ref[...] + p1_ref[...]
    hh = ts_ref[:, 0:64]
    zs = ts_ref[:, 64:68] + ta_ref[:, 0:4]
    ws = jnp.exp(jnp.maximum(zs, 0.2 * zs))
    selfagg = jnp.concatenate(
        [hh[:, h * HD:(h + 1) * HD] * ws[:, h:h + 1] for h in range(NH)],
        axis=1)
    num = agg[:, 0:64] + selfagg
    den4 = agg[:, 64:68] + ws
    den = jnp.concatenate(
        [jnp.broadcast_to(den4[:, h:h + 1], (_NB, HD)) for h in range(NH)],
        axis=1)
    out_ref[...] = jnp.maximum(num / (den + 1e-16) + bg_ref[...], 0.0)

  nspec = lambda f: pl.BlockSpec((_NB, f), lambda i: (i, 0))
  return pl.pallas_call(
      body,
      grid=(_NGRID,),
      in_specs=[nspec(80), nspec(80), nspec(80), nspec(16),
                pl.BlockSpec((1, 64), lambda i: (0, 0))],
      out_specs=nspec(64),
      out_shape=jax.ShapeDtypeStruct((NN, 64), F32),
  )(p0, p1, tsrc, tald, bg)


def _tc_pool(x_att, batch2d, wc1, bc1, wc2, bc2):
  def body(x_ref, b_ref, w1_ref, b1_ref, w2_ref, b2_ref, out_ref):
    bvec = jnp.broadcast_to(b_ref[...], (GG, NN))
    oh = (bvec == lax.broadcasted_iota(I32, (GG, NN), 0)).astype(F32)
    cnt = jnp.sum(oh, axis=1, keepdims=True)
    xg = jnp.dot(oh, x_ref[...], precision=_PREC,
                 preferred_element_type=F32) / jnp.maximum(cnt, 1.0)
    h1 = jnp.maximum(jnp.dot(xg, w1_ref[...], precision=_PREC,
                             preferred_element_type=F32) + b1_ref[...], 0.0)
    out_ref[...] = jnp.dot(h1, w2_ref[...], precision=_PREC,
                           preferred_element_type=F32) + b2_ref[...]

  return pl.pallas_call(
      body,
      out_shape=jax.ShapeDtypeStruct((GG, 2), F32),
  )(x_att, batch2d, wc1, bc1, wc2, bc2)


def kernel(x, edge_index, batch, W1, b1, W2, b2, W3, b3, Wg, a_src, a_dst,
           bg, Wc1, bc1, Wc2, bc2):
  src = edge_index[0]
  dst = edge_index[1]
  pad = jnp.arange(EPAD - EE, dtype=I32)
  src1d = jnp.concatenate([src, pad % NN])
  dst1d = jnp.concatenate([dst, NN + pad % (NACC - NN)])

  cntp = _seg_scatter("const", HD, jnp.zeros((8, 128), F32), src1d, dst1d)
  cnt = cntp[:, :NN]
  t1, dinv = _tc1(x, cnt[0], cnt[1], W1)

  p1 = _seg_scatter("gather", HID, t1, src1d, dst1d)[:, :NN]
  y1, t2 = _tc_layer(p1[0], p1[1], t1, dinv, b1.reshape(1, HID), W2,
                     None, HID, HID)
  p2 = _seg_scatter("gather", HID, t2, src1d, dst1d)[:, :NN]
  y2, t3 = _tc_layer(p2[0], p2[1], t2, dinv, b2.reshape(1, HID), W3,
                     y1, HID, HID // 2)
  p3 = _seg_scatter("gather", HID // 2, t3, src1d, dst1d)[:, :NN]

  rep = jnp.repeat(jnp.arange(NH), HD)
  asrc = jnp.zeros((NH * HD, NH), F32).at[jnp.arange(NH * HD), rep].set(
      a_src.reshape(-1))
  adst = jnp.zeros((NH * HD, NH), F32).at[jnp.arange(NH * HD), rep].set(
      a_dst.reshape(-1))
  tsrc, tald = _tc_gatprep(p3[0], p3[1], t3, dinv, b3.reshape(1, HID // 2),
                           Wg, asrc, adst)

  gsrc, gz = _gat_gather(tsrc, tald, src1d, dst1d)
  sdata = _tc_edge(gsrc, gz)
  pgat = _seg_scatter("linear", 80, sdata, src1d, dst1d)[:, :NN]

  x_att = _tc_att(pgat[0], pgat[1], tsrc, tald, bg.reshape(1, NH * HD))
  return _tc_pool(x_att, batch.reshape(1, NN), Wc1, bc1.reshape(1, HID // 2),
                  Wc2, bc2.reshape(1, 2))


# speedup vs baseline: 44.5485x; 1.5921x over previous
import functools

import jax
import jax.numpy as jnp
from jax import lax
from jax.experimental import pallas as pl
from jax.experimental.pallas import tpu as pltpu
from jax.experimental.pallas import tpu_sc as plsc

F32 = jnp.float32
I32 = jnp.int32

NN = 10000
EE = 320000
DD = 128
HID = 64
GG = 16
NH = 4
HD = 16

NC = 2
NS = 16
NW = NC * NS
CH = 128
NB = 4
CPW = 80
EPAD = NW * CPW * CH
OUTER = CPW // NB
NACC = 10240
ZR = NACC // NS
WR = NN // NS

_MESH = plsc.VectorSubcoreMesh(core_axis_name="c", subcore_axis_name="s")
_SC_PARAMS = pltpu.CompilerParams(use_tc_tiling_on_sc=False)
_PREC = lax.Precision.HIGHEST
_NGRID = 5
_NB = NN // _NGRID


def _seg_scatter(mode, feat, table, src3, dst3):
  zeros = jnp.zeros((NACC, feat), F32)

  scratch = [
      pltpu.VMEM((CPW, CH), I32),
      pltpu.VMEM((CPW, CH), I32),
      pltpu.VMEM((NB, CH, feat), F32),
      pltpu.VMEM_SHARED((NACC, feat), F32),
  ] + [pltpu.SemaphoreType.DMA] * (2 * NB)

  @functools.partial(
      pl.kernel,
      out_type=jax.ShapeDtypeStruct((NC, NACC, feat), F32),
      mesh=_MESH,
      scratch_types=scratch,
      compiler_params=_SC_PARAMS,
      name=f"sc_seg_scatter_{mode}_{feat}",
  )
  def run(table_hbm, src_hbm, dst_hbm, zeros_hbm, out_hbm, sall, dall, rows,
          acc, *sems):
    gs, ss = sems[:NB], sems[NB:]
    c = lax.axis_index("c")
    s = lax.axis_index("s")
    w = c * NS + s

    pltpu.sync_copy(zeros_hbm.at[pl.ds(s * ZR, ZR)],
                    acc.at[pl.ds(s * ZR, ZR)])
    pltpu.sync_copy(dst_hbm.at[w], dall)
    if mode == "gather":
      pltpu.sync_copy(src_hbm.at[w], sall)
    if mode == "const":
      for k in range(CH):
        rows[0, k, :] = jnp.full((feat,), 1.0, F32)
    plsc.subcore_barrier()

    def gather_copy(b, j):
      if mode == "gather":
        return pltpu.make_async_copy(table_hbm.at[sall.at[j]], rows.at[b],
                                     gs[b])
      return pltpu.make_async_copy(
          table_hbm.at[pl.ds((w * CPW + j) * CH, CH)], rows.at[b], gs[b])

    def scat_copy(b, j):
      rb = 0 if mode == "const" else b
      return pltpu.make_async_copy(rows.at[rb], acc.at[dall.at[j]], ss[b])

    def start_scat(b, j):
      rb = 0 if mode == "const" else b
      pltpu.async_copy(rows.at[rb], acc.at[dall.at[j]], ss[b], add=True)

    if mode == "const":
      @pl.loop(0, OUTER)
      def _(jo):
        for b in range(NB):
          start_scat(b, jo * NB + b)
        for b in range(NB):
          scat_copy(b, jo * NB + b).wait()
    else:
      for b in range(NB):
        gather_copy(b, b).start()

      @pl.loop(0, OUTER - 1)
      def _(jo):
        for b in range(NB):
          j = jo * NB + b
          gather_copy(b, j).wait()
          start_scat(b, j)
          scat_copy(b, j).wait()
          gather_copy(b, j + NB).start()

      for b in range(NB):
        j = (OUTER - 1) * NB + b
        gather_copy(b, j).wait()
        start_scat(b, j)
        scat_copy(b, j).wait()

    plsc.subcore_barrier()
    pltpu.sync_copy(acc.at[pl.ds(s * ZR, ZR)],
                    out_hbm.at[c, pl.ds(s * ZR, ZR)])

  return run(table, src3, dst3, zeros)


def _gat_gather(tsrc, tald, src3, dst3):
  scratch = [
      pltpu.VMEM((CPW, CH), I32),
      pltpu.VMEM((CPW, CH), I32),
      pltpu.VMEM((NB, CH, 80), F32),
      pltpu.VMEM((NB, CH, 16), F32),
  ] + [pltpu.SemaphoreType.DMA] * (3 * NB)

  @functools.partial(
      pl.kernel,
      out_type=(jax.ShapeDtypeStruct((EPAD, 80), F32),
                jax.ShapeDtypeStruct((EPAD, 16), F32)),
      mesh=_MESH,
      scratch_types=scratch,
      compiler_params=_SC_PARAMS,
      name="sc_gat_gather",
  )
  def run(tsrc_hbm, tald_hbm, src_hbm, dst_hbm, gsrc_hbm, gz_hbm, sall, dall,
          rows80, rows16, *sems):
    gs, hs, ws = sems[:NB], sems[NB:2 * NB], sems[2 * NB:]
    c = lax.axis_index("c")
    s = lax.axis_index("s")
    w = c * NS + s

    pltpu.sync_copy(src_hbm.at[w], sall)
    pltpu.sync_copy(dst_hbm.at[w], dall)

    def g80(b, j):
      return pltpu.make_async_copy(tsrc_hbm.at[sall.at[j]], rows80.at[b],
                                   gs[b])

    def g16(b, j):
      return pltpu.make_async_copy(tald_hbm.at[dall.at[j]], rows16.at[b],
                                   hs[b])

    def w80(b, j):
      r = w * CPW + j
      return pltpu.make_async_copy(rows80.at[b], gsrc_hbm.at[pl.ds(r * CH, CH)],
                                   ws[b])

    def w16(b, j):
      r = w * CPW + j
      return pltpu.make_async_copy(rows16.at[b], gz_hbm.at[pl.ds(r * CH, CH)],
                                   ws[b])

    for b in range(NB):
      g80(b, b).start()
      g16(b, b).start()

    @pl.loop(0, OUTER - 1)
    def _(jo):
      for b in range(NB):
        j = jo * NB + b
        g80(b, j).wait()
        g16(b, j).wait()
        w80(b, j).start()
        w16(b, j).start()
        w80(b, j).wait()
        w16(b, j).wait()
        g80(b, j + NB).start()
        g16(b, j + NB).start()

    for b in range(NB):
      j = (OUTER - 1) * NB + b
      g80(b, j).wait()
      g16(b, j).wait()
      w80(b, j).start()
      w16(b, j).start()
      w80(b, j).wait()
      w16(b, j).wait()

  return run(tsrc, tald, src3, dst3)


def _tc1(x, c0, c1, w1):
  def body(x_ref, c0_ref, c1_ref, w_ref, t_ref, dinv_ref):
    deg = 1.0 + c0_ref[:, 0:1] + c1_ref[:, 0:1]
    dinv = lax.rsqrt(deg)
    h = jnp.dot(x_ref[...], w_ref[...], precision=_PREC,
                preferred_element_type=F32)
    t_ref[...] = dinv * h
    dinv_ref[...] = dinv

  return pl.pallas_call(
      body,
      grid=(_NGRID,),
      in_specs=[pl.BlockSpec((_NB, DD), lambda i: (i, 0)),
                pl.BlockSpec((_NB, HD), lambda i: (i, 0)),
                pl.BlockSpec((_NB, HD), lambda i: (i, 0)),
                pl.BlockSpec((DD, HID), lambda i: (0, 0))],
      out_specs=(pl.BlockSpec((_NB, HID), lambda i: (i, 0)),
                 pl.BlockSpec((_NB, 1), lambda i: (i, 0))),
      out_shape=(jax.ShapeDtypeStruct((NN, HID), F32),
                 jax.ShapeDtypeStruct((NN, 1), F32)),
  )(x, c0, c1, w1)


def _tc_layer(p0, p1, tprev, dinv, b, wnext, res, fin, fout):
  def body(*refs):
    if res is None:
      p0_ref, p1_ref, t_ref, d_ref, b_ref, w_ref, y_ref, tn_ref = refs
    else:
      p0_ref, p1_ref, t_ref, d_ref, b_ref, w_ref, r_ref, y_ref, tn_ref = refs
    dinv_v = d_ref[...]
    y = jnp.maximum(dinv_v * (p0_ref[...] + p1_ref[...] + t_ref[...])
                    + b_ref[...], 0.0)
    if res is not None:
      y = y + r_ref[...]
    y_ref[...] = y
    tn_ref[...] = dinv_v * jnp.dot(y, w_ref[...], precision=_PREC,
                                   preferred_element_type=F32)

  args = [p0, p1, tprev, dinv, b, wnext] + ([] if res is None else [res])
  nspec = lambda f: pl.BlockSpec((_NB, f), lambda i: (i, 0))
  wspec = lambda a, bb: pl.BlockSpec((a, bb), lambda i: (0, 0))
  in_specs = [nspec(fin), nspec(fin), nspec(fin), nspec(1), wspec(1, fin),
              wspec(fin, fout)] + ([] if res is None else [nspec(fin)])
  return pl.pallas_call(
      body,
      grid=(_NGRID,),
      in_specs=in_specs,
      out_specs=(nspec(fin), nspec(fout)),
      out_shape=(jax.ShapeDtypeStruct((NN, fin), F32),
                 jax.ShapeDtypeStruct((NN, fout), F32)),
  )(*args)


def _tc_gatprep(p0, p1, t3, dinv, b3, wg, asrc, adst):
  def body(p0_ref, p1_ref, t_ref, d_ref, b_ref, wg_ref, as_ref, ad_ref,
           tsrc_ref, tald_ref):
    x3 = jnp.maximum(d_ref[...] * (p0_ref[...] + p1_ref[...] + t_ref[...])
                     + b_ref[...], 0.0)
    hh = jnp.dot(x3, wg_ref[...], precision=_PREC, preferred_element_type=F32)
    als = jnp.dot(hh, as_ref[...], precision=_PREC, preferred_element_type=F32)
    ald = jnp.dot(hh, ad_ref[...], precision=_PREC, preferred_element_type=F32)
    pad = jnp.zeros((_NB, 12), F32)
    tsrc_ref[...] = jnp.concatenate([hh, als, pad], axis=1)
    tald_ref[...] = jnp.concatenate([ald, pad], axis=1)

  nspec = lambda f: pl.BlockSpec((_NB, f), lambda i: (i, 0))
  wspec = lambda a, bb: pl.BlockSpec((a, bb), lambda i: (0, 0))
  return pl.pallas_call(
      body,
      grid=(_NGRID,),
      in_specs=[nspec(32), nspec(32), nspec(32), nspec(1), wspec(1, 32),
                wspec(32, 64), wspec(64, NH), wspec(64, NH)],
      out_specs=(nspec(80), nspec(16)),
      out_shape=(jax.ShapeDtypeStruct((NN, 80), F32),
                 jax.ShapeDtypeStruct((NN, 16), F32)),
  )(p0, p1, t3, dinv, b3, wg, asrc, adst)


_EGRID = 32
_EB = EPAD // _EGRID


def _tc_edge(gsrc, gz):
  def body(g_ref, z_ref, s_ref):
    g = g_ref[...]
    z = g[:, 64:80] + z_ref[...]
    w16 = jnp.exp(jnp.maximum(z, 0.2 * z))
    parts = [g[:, h * HD:(h + 1) * HD] * w16[:, h:h + 1] for h in range(NH)]
    s_ref[...] = jnp.concatenate(parts + [w16], axis=1)

  return pl.pallas_call(
      body,
      grid=(_EGRID,),
      in_specs=[pl.BlockSpec((_EB, 80), lambda i: (i, 0)),
                pl.BlockSpec((_EB, 16), lambda i: (i, 0))],
      out_specs=pl.BlockSpec((_EB, 80), lambda i: (i, 0)),
      out_shape=jax.ShapeDtypeStruct((EPAD, 80), F32),
  )(gsrc, gz)


def _tc_att(p0, p1, tsrc, tald, bg):
  def body(p0_ref, p1_ref, ts_ref, ta_ref, bg_ref, out_ref):
    agg = p0_ref[...] + p1_ref[...]
    hh = ts_ref[:, 0:64]
    zs = ts_ref[:, 64:68] + ta_ref[:, 0:4]
    ws = jnp.exp(jnp.maximum(zs, 0.2 * zs))
    selfagg = jnp.concatenate(
        [hh[:, h * HD:(h + 1) * HD] * ws[:, h:h + 1] for h in range(NH)],
        axis=1)
    num = agg[:, 0:64] + selfagg
    den4 = agg[:, 64:68] + ws
    den = jnp.concatenate(
        [jnp.broadcast_to(den4[:, h:h + 1], (_NB, HD)) for h in range(NH)],
        axis=1)
    out_ref[...] = jnp.maximum(num / (den + 1e-16) + bg_ref[...], 0.0)

  nspec = lambda f: pl.BlockSpec((_NB, f), lambda i: (i, 0))
  return pl.pallas_call(
      body,
      grid=(_NGRID,),
      in_specs=[nspec(80), nspec(80), nspec(80), nspec(16),
                pl.BlockSpec((1, 64), lambda i: (0, 0))],
      out_specs=nspec(64),
      out_shape=jax.ShapeDtypeStruct((NN, 64), F32),
  )(p0, p1, tsrc, tald, bg)


def _tc_pool(x_att, batch2d, wc1, bc1, wc2, bc2):
  def body(x_ref, b_ref, w1_ref, b1_ref, w2_ref, b2_ref, out_ref):
    bvec = jnp.broadcast_to(b_ref[...], (GG, NN))
    oh = (bvec == lax.broadcasted_iota(I32, (GG, NN), 0)).astype(F32)
    cnt = jnp.sum(oh, axis=1, keepdims=True)
    xg = jnp.dot(oh, x_ref[...], precision=_PREC,
                 preferred_element_type=F32) / jnp.maximum(cnt, 1.0)
    h1 = jnp.maximum(jnp.dot(xg, w1_ref[...], precision=_PREC,
                             preferred_element_type=F32) + b1_ref[...], 0.0)
    out_ref[...] = jnp.dot(h1, w2_ref[...], precision=_PREC,
                           preferred_element_type=F32) + b2_ref[...]

  return pl.pallas_call(
      body,
      out_shape=jax.ShapeDtypeStruct((GG, 2), F32),
  )(x_att, batch2d, wc1, bc1, wc2, bc2)


def kernel(x, edge_index, batch, W1, b1, W2, b2, W3, b3, Wg, a_src, a_dst,
           bg, Wc1, bc1, Wc2, bc2):
  src = edge_index[0]
  dst = edge_index[1]
  pad = jnp.arange(EPAD - EE, dtype=I32)
  src3 = jnp.concatenate([src, pad % NN]).reshape(NW, CPW, CH)
  dst3 = jnp.concatenate([dst, NN + pad % (NACC - NN)]).reshape(NW, CPW, CH)

  cntp = _seg_scatter("const", HD, jnp.zeros((8, 128), F32), src3, dst3)
  cnt = cntp[:, :NN]
  t1, dinv = _tc1(x, cnt[0], cnt[1], W1)

  p1 = _seg_scatter("gather", HID, t1, src3, dst3)[:, :NN]
  y1, t2 = _tc_layer(p1[0], p1[1], t1, dinv, b1.reshape(1, HID), W2,
                     None, HID, HID)
  p2 = _seg_scatter("gather", HID, t2, src3, dst3)[:, :NN]
  y2, t3 = _tc_layer(p2[0], p2[1], t2, dinv, b2.reshape(1, HID), W3,
                     y1, HID, HID // 2)
  p3 = _seg_scatter("gather", HID // 2, t3, src3, dst3)[:, :NN]

  rep = jnp.repeat(jnp.arange(NH), HD)
  asrc = jnp.zeros((NH * HD, NH), F32).at[jnp.arange(NH * HD), rep].set(
      a_src.reshape(-1))
  adst = jnp.zeros((NH * HD, NH), F32).at[jnp.arange(NH * HD), rep].set(
      a_dst.reshape(-1))
  tsrc, tald = _tc_gatprep(p3[0], p3[1], t3, dinv, b3.reshape(1, HID // 2),
                           Wg, asrc, adst)

  gsrc, gz = _gat_gather(tsrc, tald, src3, dst3)
  sdata = _tc_edge(gsrc, gz)
  pgat = _seg_scatter("linear", 80, sdata, src3, dst3)[:, :NN]

  x_att = _tc_att(pgat[0], pgat[1], tsrc, tald, bg.reshape(1, NH * HD))
  return _tc_pool(x_att, batch.reshape(1, NN), Wc1, bc1.reshape(1, HID // 2),
                  Wc2, bc2.reshape(1, 2))


# speedup vs baseline: 101.9626x; 2.2888x over previous
import functools

import jax
import jax.numpy as jnp
from jax import lax
from jax.experimental import pallas as pl
from jax.experimental.pallas import tpu as pltpu
from jax.experimental.pallas import tpu_sc as plsc

F32 = jnp.float32
I32 = jnp.int32

NN = 10000
EE = 320000
DD = 128
HID = 64
GG = 16
NH = 4
HD = 16

NC = 2
NS = 16
NW = NC * NS
CH = 128
NB = 4
CPW = 80
EPAD = NW * CPW * CH
OUTER = CPW // NB
NACC = 10240
ZR = NACC // NS
WR = NN // NS

_MESH = plsc.VectorSubcoreMesh(core_axis_name="c", subcore_axis_name="s")
_SC_PARAMS = pltpu.CompilerParams(use_tc_tiling_on_sc=False)
_PREC = lax.Precision.HIGHEST
_NGRID = 5
_NB = NN // _NGRID


def _seg_scatter(mode, feat, table, src3, dst3):
  zeros = jnp.zeros((NACC, feat), F32)

  scratch = [
      pltpu.VMEM((CPW, CH), I32),
      pltpu.VMEM((CPW, CH), I32),
      pltpu.VMEM((NB, CH, feat), F32),
      pltpu.VMEM_SHARED((NACC, feat), F32),
  ] + [pltpu.SemaphoreType.DMA] * (2 * NB)

  @functools.partial(
      pl.kernel,
      out_type=jax.ShapeDtypeStruct((NC, NACC, feat), F32),
      mesh=_MESH,
      scratch_types=scratch,
      compiler_params=_SC_PARAMS,
      name=f"sc_seg_scatter_{mode}_{feat}",
  )
  def run(table_hbm, src_hbm, dst_hbm, zeros_hbm, out_hbm, sall, dall, rows,
          acc, *sems):
    gs, ss = sems[:NB], sems[NB:]
    c = lax.axis_index("c")
    s = lax.axis_index("s")
    w = c * NS + s

    pltpu.sync_copy(zeros_hbm.at[pl.ds(s * ZR, ZR)],
                    acc.at[pl.ds(s * ZR, ZR)])
    pltpu.sync_copy(dst_hbm.at[w], dall)
    if mode == "gather":
      pltpu.sync_copy(src_hbm.at[w], sall)
    if mode == "const":
      for k in range(CH):
        rows[0, k, :] = jnp.full((feat,), 1.0, F32)
    plsc.subcore_barrier()

    def gather_copy(b, j):
      if mode == "gather":
        return pltpu.make_async_copy(table_hbm.at[sall.at[j]], rows.at[b],
                                     gs[b])
      return pltpu.make_async_copy(
          table_hbm.at[pl.ds((w * CPW + j) * CH, CH)], rows.at[b], gs[b])

    def scat_copy(b, j):
      rb = 0 if mode == "const" else b
      return pltpu.make_async_copy(rows.at[rb], acc.at[dall.at[j]], ss[b])

    def start_scat(b, j):
      rb = 0 if mode == "const" else b
      pltpu.async_copy(rows.at[rb], acc.at[dall.at[j]], ss[b], add=True)

    if mode == "const":
      @pl.loop(0, OUTER)
      def _(jo):
        for b in range(NB):
          start_scat(b, jo * NB + b)
        for b in range(NB):
          scat_copy(b, jo * NB + b).wait()
    else:
      for b in range(NB):
        gather_copy(b, b).start()

      @pl.loop(0, OUTER - 1)
      def _(jo):
        for b in range(NB):
          j = jo * NB + b
          gather_copy(b, j).wait()
          start_scat(b, j)
          scat_copy(b, j).wait()
          gather_copy(b, j + NB).start()

      for b in range(NB):
        j = (OUTER - 1) * NB + b
        gather_copy(b, j).wait()
        start_scat(b, j)
        scat_copy(b, j).wait()

    plsc.subcore_barrier()
    pltpu.sync_copy(acc.at[pl.ds(s * ZR, ZR)],
                    out_hbm.at[c, pl.ds(s * ZR, ZR)])

  return run(table, src3, dst3, zeros)


_NB2 = 2


def _gat_fused(tsrc, tald, src3, dst3):
  zeros = jnp.zeros((NACC, 80), F32)

  scratch = [
      pltpu.VMEM((CPW, CH), I32),
      pltpu.VMEM((CPW, CH), I32),
      pltpu.VMEM((_NB2, CH, 80), F32),
      pltpu.VMEM((_NB2, CH, 16), F32),
      pltpu.VMEM_SHARED((NACC, 80), F32),
  ] + [pltpu.SemaphoreType.DMA] * (3 * _NB2)

  @functools.partial(
      pl.kernel,
      out_type=jax.ShapeDtypeStruct((NC, NACC, 80), F32),
      mesh=_MESH,
      scratch_types=scratch,
      compiler_params=_SC_PARAMS,
      name="sc_gat_fused",
  )
  def run(tsrc_hbm, tald_hbm, src_hbm, dst_hbm, zeros_hbm, out_hbm, sall,
          dall, rows80, rows16, acc, *sems):
    gs, hs, ss = sems[:_NB2], sems[_NB2:2 * _NB2], sems[2 * _NB2:]
    c = lax.axis_index("c")
    s = lax.axis_index("s")
    w = c * NS + s

    pltpu.sync_copy(zeros_hbm.at[pl.ds(s * ZR, ZR)],
                    acc.at[pl.ds(s * ZR, ZR)])
    pltpu.sync_copy(src_hbm.at[w], sall)
    pltpu.sync_copy(dst_hbm.at[w], dall)
    plsc.subcore_barrier()

    def g80(b, j):
      return pltpu.make_async_copy(tsrc_hbm.at[sall.at[j]], rows80.at[b],
                                   gs[b])

    def g16(b, j):
      return pltpu.make_async_copy(tald_hbm.at[dall.at[j]], rows16.at[b],
                                   hs[b])

    def scat(b, j):
      return pltpu.make_async_copy(rows80.at[b], acc.at[dall.at[j]], ss[b])

    def compute(b):
      r80 = rows80.at[b]
      r16 = rows16.at[b]
      for k in range(CH):
        z = r80[k, pl.ds(64, HD)] + r16[k, :]
        wv = jnp.exp(jnp.maximum(z, 0.2 * z))
        r80[k, pl.ds(64, HD)] = wv
        for h in range(NH):
          bc = jnp.take(wv, jnp.full((HD,), h, I32))
          r80[k, pl.ds(h * HD, HD)] = r80[k, pl.ds(h * HD, HD)] * bc

    for b in range(_NB2):
      g80(b, b).start()
      g16(b, b).start()

    @pl.loop(0, CPW // _NB2 - 1)
    def _(jo):
      for b in range(_NB2):
        j = jo * _NB2 + b
        g80(b, j).wait()
        g16(b, j).wait()
        compute(b)
        pltpu.async_copy(rows80.at[b], acc.at[dall.at[j]], ss[b], add=True)
        scat(b, j).wait()
        g80(b, j + _NB2).start()
        g16(b, j + _NB2).start()

    for b in range(_NB2):
      j = CPW - _NB2 + b
      g80(b, j).wait()
      g16(b, j).wait()
      compute(b)
      pltpu.async_copy(rows80.at[b], acc.at[dall.at[j]], ss[b], add=True)
      scat(b, j).wait()

    plsc.subcore_barrier()
    pltpu.sync_copy(acc.at[pl.ds(s * ZR, ZR)],
                    out_hbm.at[c, pl.ds(s * ZR, ZR)])

  return run(tsrc, tald, src3, dst3, zeros)


def _tc1(x, c0, c1, w1):
  def body(x_ref, c0_ref, c1_ref, w_ref, t_ref, dinv_ref):
    deg = 1.0 + c0_ref[:, 0:1] + c1_ref[:, 0:1]
    dinv = lax.rsqrt(deg)
    h = jnp.dot(x_ref[...], w_ref[...], precision=_PREC,
                preferred_element_type=F32)
    t_ref[...] = dinv * h
    dinv_ref[...] = dinv

  return pl.pallas_call(
      body,
      grid=(_NGRID,),
      in_specs=[pl.BlockSpec((_NB, DD), lambda i: (i, 0)),
                pl.BlockSpec((_NB, HD), lambda i: (i, 0)),
                pl.BlockSpec((_NB, HD), lambda i: (i, 0)),
                pl.BlockSpec((DD, HID), lambda i: (0, 0))],
      out_specs=(pl.BlockSpec((_NB, HID), lambda i: (i, 0)),
                 pl.BlockSpec((_NB, 1), lambda i: (i, 0))),
      out_shape=(jax.ShapeDtypeStruct((NN, HID), F32),
                 jax.ShapeDtypeStruct((NN, 1), F32)),
  )(x, c0, c1, w1)


def _tc_layer(p0, p1, tprev, dinv, b, wnext, res, fin, fout):
  def body(*refs):
    if res is None:
      p0_ref, p1_ref, t_ref, d_ref, b_ref, w_ref, y_ref, tn_ref = refs
    else:
      p0_ref, p1_ref, t_ref, d_ref, b_ref, w_ref, r_ref, y_ref, tn_ref = refs
    dinv_v = d_ref[...]
    y = jnp.maximum(dinv_v * (p0_ref[...] + p1_ref[...] + t_ref[...])
                    + b_ref[...], 0.0)
    if res is not None:
      y = y + r_ref[...]
    y_ref[...] = y
    tn_ref[...] = dinv_v * jnp.dot(y, w_ref[...], precision=_PREC,
                                   preferred_element_type=F32)

  args = [p0, p1, tprev, dinv, b, wnext] + ([] if res is None else [res])
  nspec = lambda f: pl.BlockSpec((_NB, f), lambda i: (i, 0))
  wspec = lambda a, bb: pl.BlockSpec((a, bb), lambda i: (0, 0))
  in_specs = [nspec(fin), nspec(fin), nspec(fin), nspec(1), wspec(1, fin),
              wspec(fin, fout)] + ([] if res is None else [nspec(fin)])
  return pl.pallas_call(
      body,
      grid=(_NGRID,),
      in_specs=in_specs,
      out_specs=(nspec(fin), nspec(fout)),
      out_shape=(jax.ShapeDtypeStruct((NN, fin), F32),
                 jax.ShapeDtypeStruct((NN, fout), F32)),
  )(*args)


def _tc_gatprep(p0, p1, t3, dinv, b3, wg, asrc, adst):
  def body(p0_ref, p1_ref, t_ref, d_ref, b_ref, wg_ref, as_ref, ad_ref,
           tsrc_ref, tald_ref):
    x3 = jnp.maximum(d_ref[...] * (p0_ref[...] + p1_ref[...] + t_ref[...])
                     + b_ref[...], 0.0)
    hh = jnp.dot(x3, wg_ref[...], precision=_PREC, preferred_element_type=F32)
    als = jnp.dot(hh, as_ref[...], precision=_PREC, preferred_element_type=F32)
    ald = jnp.dot(hh, ad_ref[...], precision=_PREC, preferred_element_type=F32)
    pad = jnp.zeros((_NB, 12), F32)
    tsrc_ref[...] = jnp.concatenate([hh, als, pad], axis=1)
    tald_ref[...] = jnp.concatenate([ald, pad], axis=1)

  nspec = lambda f: pl.BlockSpec((_NB, f), lambda i: (i, 0))
  wspec = lambda a, bb: pl.BlockSpec((a, bb), lambda i: (0, 0))
  return pl.pallas_call(
      body,
      grid=(_NGRID,),
      in_specs=[nspec(32), nspec(32), nspec(32), nspec(1), wspec(1, 32),
                wspec(32, 64), wspec(64, NH), wspec(64, NH)],
      out_specs=(nspec(80), nspec(16)),
      out_shape=(jax.ShapeDtypeStruct((NN, 80), F32),
                 jax.ShapeDtypeStruct((NN, 16), F32)),
  )(p0, p1, t3, dinv, b3, wg, asrc, adst)


def _tc_att(p0, p1, tsrc, tald, bg):
  def body(p0_ref, p1_ref, ts_ref, ta_ref, bg_ref, out_ref):
    agg = p0_ref[...] + p1_ref[...]
    hh = ts_ref[:, 0:64]
    zs = ts_ref[:, 64:68] + ta_ref[:, 0:4]
    ws = jnp.exp(jnp.maximum(zs, 0.2 * zs))
    selfagg = jnp.concatenate(
        [hh[:, h * HD:(h + 1) * HD] * ws[:, h:h + 1] for h in range(NH)],
        axis=1)
    num = agg[:, 0:64] + selfagg
    den4 = agg[:, 64:68] + ws
    den = jnp.concatenate(
        [jnp.broadcast_to(den4[:, h:h + 1], (_NB, HD)) for h in range(NH)],
        axis=1)
    out_ref[...] = jnp.maximum(num / (den + 1e-16) + bg_ref[...], 0.0)

  nspec = lambda f: pl.BlockSpec((_NB, f), lambda i: (i, 0))
  return pl.pallas_call(
      body,
      grid=(_NGRID,),
      in_specs=[nspec(80), nspec(80), nspec(80), nspec(16),
                pl.BlockSpec((1, 64), lambda i: (0, 0))],
      out_specs=nspec(64),
      out_shape=jax.ShapeDtypeStruct((NN, 64), F32),
  )(p0, p1, tsrc, tald, bg)


def _tc_pool(x_att, batch2d, wc1, bc1, wc2, bc2):
  def body(x_ref, b_ref, w1_ref, b1_ref, w2_ref, b2_ref, out_ref):
    bvec = jnp.broadcast_to(b_ref[...], (GG, NN))
    oh = (bvec == lax.broadcasted_iota(I32, (GG, NN), 0)).astype(F32)
    cnt = jnp.sum(oh, axis=1, keepdims=True)
    xg = jnp.dot(oh, x_ref[...], precision=_PREC,
                 preferred_element_type=F32) / jnp.maximum(cnt, 1.0)
    h1 = jnp.maximum(jnp.dot(xg, w1_ref[...], precision=_PREC,
                             preferred_element_type=F32) + b1_ref[...], 0.0)
    out_ref[...] = jnp.dot(h1, w2_ref[...], precision=_PREC,
                           preferred_element_type=F32) + b2_ref[...]

  return pl.pallas_call(
      body,
      out_shape=jax.ShapeDtypeStruct((GG, 2), F32),
  )(x_att, batch2d, wc1, bc1, wc2, bc2)


def kernel(x, edge_index, batch, W1, b1, W2, b2, W3, b3, Wg, a_src, a_dst,
           bg, Wc1, bc1, Wc2, bc2):
  src = edge_index[0]
  dst = edge_index[1]
  pad = jnp.arange(EPAD - EE, dtype=I32)
  src3 = jnp.concatenate([src, pad % NN]).reshape(NW, CPW, CH)
  dst3 = jnp.concatenate([dst, NN + pad % (NACC - NN)]).reshape(NW, CPW, CH)

  cntp = _seg_scatter("const", HD, jnp.zeros((8, 128), F32), src3, dst3)
  cnt = cntp[:, :NN]
  t1, dinv = _tc1(x, cnt[0], cnt[1], W1)

  p1 = _seg_scatter("gather", HID, t1, src3, dst3)[:, :NN]
  y1, t2 = _tc_layer(p1[0], p1[1], t1, dinv, b1.reshape(1, HID), W2,
                     None, HID, HID)
  p2 = _seg_scatter("gather", HID, t2, src3, dst3)[:, :NN]
  y2, t3 = _tc_layer(p2[0], p2[1], t2, dinv, b2.reshape(1, HID), W3,
                     y1, HID, HID // 2)
  p3 = _seg_scatter("gather", HID // 2, t3, src3, dst3)[:, :NN]

  rep = jnp.repeat(jnp.arange(NH), HD)
  asrc = jnp.zeros((NH * HD, NH), F32).at[jnp.arange(NH * HD), rep].set(
      a_src.reshape(-1))
  adst = jnp.zeros((NH * HD, NH), F32).at[jnp.arange(NH * HD), rep].set(
      a_dst.reshape(-1))
  tsrc, tald = _tc_gatprep(p3[0], p3[1], t3, dinv, b3.reshape(1, HID // 2),
                           Wg, asrc, adst)

  pgat = _gat_fused(tsrc, tald, src3, dst3)[:, :NN]

  x_att = _tc_att(pgat[0], pgat[1], tsrc, tald, bg.reshape(1, NH * HD))
  return _tc_pool(x_att, batch.reshape(1, NN), Wc1, bc1.reshape(1, HID // 2),
                  Wc2, bc2.reshape(1, 2))


# speedup vs baseline: 114.6050x; 1.1240x over previous
import functools

import jax
import jax.numpy as jnp
from jax import lax
from jax.experimental import pallas as pl
from jax.experimental.pallas import tpu as pltpu
from jax.experimental.pallas import tpu_sc as plsc

F32 = jnp.float32
I32 = jnp.int32

NN = 10000
EE = 320000
DD = 128
HID = 64
GG = 16
NH = 4
HD = 16

NC = 2
NS = 16
NW = NC * NS
CH = 128
NB = 8
PF = 4
CPW = 80
EPAD = NW * CPW * CH
NACC = 10240
ZR = NACC // NS

_MESH = plsc.VectorSubcoreMesh(core_axis_name="c", subcore_axis_name="s")
_SC_PARAMS = pltpu.CompilerParams(use_tc_tiling_on_sc=False)
_PREC = lax.Precision.HIGHEST
_NGRID = 5
_RB = NN // _NGRID


def _seg_scatter(mode, feat, table, src3, dst3):
  zeros = jnp.zeros((NACC, feat), F32)

  scratch = [
      pltpu.VMEM((CPW, CH), I32),
      pltpu.VMEM((CPW, CH), I32),
      pltpu.VMEM((NB, CH, feat), F32),
      pltpu.VMEM_SHARED((NACC, feat), F32),
  ] + [pltpu.SemaphoreType.DMA] * (2 * NB)

  @functools.partial(
      pl.kernel,
      out_type=jax.ShapeDtypeStruct((NC, NACC, feat), F32),
      mesh=_MESH,
      scratch_types=scratch,
      compiler_params=_SC_PARAMS,
      name=f"sc_seg_scatter_{mode}_{feat}",
  )
  def run(table_hbm, src_hbm, dst_hbm, zeros_hbm, out_hbm, sall, dall, rows,
          acc, *sems):
    gs, ss = sems[:NB], sems[NB:]
    c = lax.axis_index("c")
    s = lax.axis_index("s")
    w = c * NS + s

    pltpu.sync_copy(zeros_hbm.at[pl.ds(s * ZR, ZR)],
                    acc.at[pl.ds(s * ZR, ZR)])
    pltpu.sync_copy(dst_hbm.at[w], dall)
    if mode == "gather":
      pltpu.sync_copy(src_hbm.at[w], sall)
    if mode == "const":
      for k in range(CH):
        rows[0, k, :] = jnp.full((feat,), 1.0, F32)
    plsc.subcore_barrier()

    def gather(b, j):
      return pltpu.make_async_copy(table_hbm.at[sall.at[j]], rows.at[b],
                                   gs[b])

    def scat(b, j):
      rb = 0 if mode == "const" else b
      return pltpu.make_async_copy(rows.at[rb], acc.at[dall.at[j]], ss[b])

    def start_scat(b, j):
      rb = 0 if mode == "const" else b
      pltpu.async_copy(rows.at[rb], acc.at[dall.at[j]], ss[b], add=True)

    if mode == "const":
      @pl.loop(0, CPW // NB)
      def _(jo):
        for b in range(NB):
          start_scat(b, jo * NB + b)
        for b in range(NB):
          scat(b, jo * NB + b).wait()
    else:
      for i in range(PF):
        gather(i, i).start()

      @pl.loop(0, CPW // NB)
      def _(jo):
        for i in range(NB):
          j = jo * NB + i
          b = i
          bp = (i + PF) % NB
          gather(b, j).wait()
          start_scat(b, j)
          jprev = j + PF - NB

          @pl.when(jprev >= 0)
          def _():
            scat(bp, jprev).wait()

          @pl.when(j + PF < CPW)
          def _():
            gather(bp, j + PF).start()

      for i in range(NB - PF, NB):
        j = CPW - NB + i
        scat(i, j).wait()

    plsc.subcore_barrier()
    pltpu.sync_copy(acc.at[pl.ds(s * ZR, ZR)],
                    out_hbm.at[c, pl.ds(s * ZR, ZR)])

  return run(table, src3, dst3, zeros)


_NB2 = 2


def _gat_fused(tsrc, tald, src3, dst3):
  zeros = jnp.zeros((NACC, 80), F32)

  scratch = [
      pltpu.VMEM((CPW, CH), I32),
      pltpu.VMEM((CPW, CH), I32),
      pltpu.VMEM((_NB2, CH, 80), F32),
      pltpu.VMEM((_NB2, CH, 16), F32),
      pltpu.VMEM((_NB2, CH, 80), F32),
      pltpu.VMEM_SHARED((NACC, 80), F32),
  ] + [pltpu.SemaphoreType.DMA] * (3 * _NB2)

  @functools.partial(
      pl.kernel,
      out_type=jax.ShapeDtypeStruct((NC, NACC, 80), F32),
      mesh=_MESH,
      scratch_types=scratch,
      compiler_params=_SC_PARAMS,
      name="sc_gat_fused",
  )
  def run(tsrc_hbm, tald_hbm, src_hbm, dst_hbm, zeros_hbm, out_hbm, sall,
          dall, rows80, rows16, srows, acc, *sems):
    gs, hs, ss = sems[:_NB2], sems[_NB2:2 * _NB2], sems[2 * _NB2:]
    c = lax.axis_index("c")
    s = lax.axis_index("s")
    w = c * NS + s

    pltpu.sync_copy(zeros_hbm.at[pl.ds(s * ZR, ZR)],
                    acc.at[pl.ds(s * ZR, ZR)])
    pltpu.sync_copy(src_hbm.at[w], sall)
    pltpu.sync_copy(dst_hbm.at[w], dall)
    plsc.subcore_barrier()

    def g80(b, j):
      return pltpu.make_async_copy(tsrc_hbm.at[sall.at[j]], rows80.at[b],
                                   gs[b])

    def g16(b, j):
      return pltpu.make_async_copy(tald_hbm.at[dall.at[j]], rows16.at[b],
                                   hs[b])

    def scat(b, j):
      return pltpu.make_async_copy(srows.at[b], acc.at[dall.at[j]], ss[b])

    def compute(b):
      r80 = rows80.at[b]
      r16 = rows16.at[b]
      sr = srows.at[b]
      for k in range(CH):
        z = r80[k, pl.ds(64, HD)] + r16[k, :]
        wv = jnp.exp(jnp.maximum(z, 0.2 * z))
        sr[k, pl.ds(64, HD)] = wv
        for h in range(NH):
          bc = jnp.take(wv, jnp.full((HD,), h, I32))
          sr[k, pl.ds(h * HD, HD)] = r80[k, pl.ds(h * HD, HD)] * bc

    for b in range(_NB2):
      g80(b, b).start()
      g16(b, b).start()

    @pl.loop(0, CPW // _NB2)
    def _(jo):
      for b in range(_NB2):
        j = jo * _NB2 + b
        g80(b, j).wait()
        g16(b, j).wait()

        @pl.when(jo > 0)
        def _():
          scat(b, j - _NB2).wait()

        compute(b)
        pltpu.async_copy(srows.at[b], acc.at[dall.at[j]], ss[b], add=True)

        @pl.when(j + _NB2 < CPW)
        def _():
          g80(b, j + _NB2).start()
          g16(b, j + _NB2).start()

    for b in range(_NB2):
      scat(b, CPW - _NB2 + b).wait()

    plsc.subcore_barrier()
    pltpu.sync_copy(acc.at[pl.ds(s * ZR, ZR)],
                    out_hbm.at[c, pl.ds(s * ZR, ZR)])

  return run(tsrc, tald, src3, dst3, zeros)


def _pspec(f):
  return (pl.BlockSpec((1, _RB, f), lambda i: (0, i, 0)),
          pl.BlockSpec((1, _RB, f), lambda i: (1, i, 0)))


def _nspec(f):
  return pl.BlockSpec((_RB, f), lambda i: (i, 0))


def _wspec(a, b):
  return pl.BlockSpec((a, b), lambda i: (0, 0))


def _tc1(x, cntp, w1):
  def body(x_ref, c0_ref, c1_ref, w_ref, t_ref, dinv_ref):
    deg = 1.0 + c0_ref[0, :, 0:1] + c1_ref[0, :, 0:1]
    dinv = lax.rsqrt(deg)
    h = jnp.dot(x_ref[...], w_ref[...], precision=_PREC,
                preferred_element_type=F32)
    t_ref[...] = dinv * h
    dinv_ref[...] = dinv

  p0s, p1s = _pspec(HD)
  return pl.pallas_call(
      body,
      grid=(_NGRID,),
      in_specs=[_nspec(DD), p0s, p1s, _wspec(DD, HID)],
      out_specs=(_nspec(HID), _nspec(1)),
      out_shape=(jax.ShapeDtypeStruct((NN, HID), F32),
                 jax.ShapeDtypeStruct((NN, 1), F32)),
  )(x, cntp, cntp, w1)


def _tc_layer(p, tprev, dinv, b, wnext, res, fin, fout):
  def body(*refs):
    if res is None:
      p0_ref, p1_ref, t_ref, d_ref, b_ref, w_ref, y_ref, tn_ref = refs
    else:
      p0_ref, p1_ref, t_ref, d_ref, b_ref, w_ref, r_ref, y_ref, tn_ref = refs
    dinv_v = d_ref[...]
    y = jnp.maximum(dinv_v * (p0_ref[0] + p1_ref[0] + t_ref[...])
                    + b_ref[...], 0.0)
    if res is not None:
      y = y + r_ref[...]
    y_ref[...] = y
    tn_ref[...] = dinv_v * jnp.dot(y, w_ref[...], precision=_PREC,
                                   preferred_element_type=F32)

  p0s, p1s = _pspec(fin)
  args = [p, p, tprev, dinv, b, wnext] + ([] if res is None else [res])
  in_specs = [p0s, p1s, _nspec(fin), _nspec(1), _wspec(1, fin),
              _wspec(fin, fout)] + ([] if res is None else [_nspec(fin)])
  return pl.pallas_call(
      body,
      grid=(_NGRID,),
      in_specs=in_specs,
      out_specs=(_nspec(fin), _nspec(fout)),
      out_shape=(jax.ShapeDtypeStruct((NN, fin), F32),
                 jax.ShapeDtypeStruct((NN, fout), F32)),
  )(*args)


def _tc_gatprep(p, t3, dinv, b3, wg, asrc, adst):
  def body(p0_ref, p1_ref, t_ref, d_ref, b_ref, wg_ref, as_ref, ad_ref,
           tsrc_ref, tald_ref):
    x3 = jnp.maximum(d_ref[...] * (p0_ref[0] + p1_ref[0] + t_ref[...])
                     + b_ref[...], 0.0)
    hh = jnp.dot(x3, wg_ref[...], precision=_PREC, preferred_element_type=F32)
    als = jnp.dot(hh, as_ref[...], precision=_PREC, preferred_element_type=F32)
    ald = jnp.dot(hh, ad_ref[...], precision=_PREC, preferred_element_type=F32)
    pad = jnp.zeros((_RB, 12), F32)
    tsrc_ref[...] = jnp.concatenate([hh, als, pad], axis=1)
    tald_ref[...] = jnp.concatenate([ald, pad], axis=1)

  p0s, p1s = _pspec(32)
  return pl.pallas_call(
      body,
      grid=(_NGRID,),
      in_specs=[p0s, p1s, _nspec(32), _nspec(1), _wspec(1, 32),
                _wspec(32, 64), _wspec(64, NH), _wspec(64, NH)],
      out_specs=(_nspec(80), _nspec(16)),
      out_shape=(jax.ShapeDtypeStruct((NN, 80), F32),
                 jax.ShapeDtypeStruct((NN, 16), F32)),
  )(p, p, t3, dinv, b3, wg, asrc, adst)


def _tc_att_pool(p, tsrc, tald, bg, batch2d, wc1, bc1, wc2, bc2):
  def body(p0_ref, p1_ref, ts_ref, ta_ref, bg_ref, b_ref, w1_ref, b1_ref,
           w2_ref, b2_ref, out_ref, xg_acc, cnt_acc):
    i = pl.program_id(0)
    agg = p0_ref[0] + p1_ref[0]
    hh = ts_ref[:, 0:64]
    zs = ts_ref[:, 64:68] + ta_ref[:, 0:4]
    ws = jnp.exp(jnp.maximum(zs, 0.2 * zs))
    selfagg = jnp.concatenate(
        [hh[:, h * HD:(h + 1) * HD] * ws[:, h:h + 1] for h in range(NH)],
        axis=1)
    num = agg[:, 0:64] + selfagg
    den4 = agg[:, 64:68] + ws
    den = jnp.concatenate(
        [jnp.broadcast_to(den4[:, h:h + 1], (_RB, HD)) for h in range(NH)],
        axis=1)
    x_att = jnp.maximum(num / (den + 1e-16) + bg_ref[...], 0.0)

    oh = (jnp.broadcast_to(b_ref[...], (_RB, GG))
          == lax.broadcasted_iota(I32, (_RB, GG), 1)).astype(F32)
    xg_blk = lax.dot_general(oh, x_att, (((0,), (0,)), ((), ())),
                             precision=_PREC, preferred_element_type=F32)
    cnt_blk = lax.dot_general(oh, jnp.ones((_RB, 1), F32),
                              (((0,), (0,)), ((), ())), precision=_PREC,
                              preferred_element_type=F32)

    @pl.when(i == 0)
    def _():
      xg_acc[...] = jnp.zeros((GG, HID), F32)
      cnt_acc[...] = jnp.zeros((GG, 1), F32)

    xg_acc[...] += xg_blk
    cnt_acc[...] += cnt_blk

    @pl.when(i == _NGRID - 1)
    def _():
      xg = xg_acc[...] / jnp.maximum(cnt_acc[...], 1.0)
      h1 = jnp.maximum(jnp.dot(xg, w1_ref[...], precision=_PREC,
                               preferred_element_type=F32) + b1_ref[...], 0.0)
      out_ref[...] = jnp.dot(h1, w2_ref[...], precision=_PREC,
                             preferred_element_type=F32) + b2_ref[...]

  p0s, p1s = _pspec(80)
  return pl.pallas_call(
      body,
      grid=(_NGRID,),
      in_specs=[p0s, p1s, _nspec(80), _nspec(16), _wspec(1, 64),
                _nspec(1), _wspec(HID, 32),
                _wspec(1, 32), _wspec(32, 2), _wspec(1, 2)],
      out_specs=pl.BlockSpec((GG, 2), lambda i: (0, 0)),
      out_shape=jax.ShapeDtypeStruct((GG, 2), F32),
      scratch_shapes=[pltpu.VMEM((GG, HID), F32), pltpu.VMEM((GG, 1), F32)],
  )(p, p, tsrc, tald, bg, batch2d, wc1, bc1, wc2, bc2)


def kernel(x, edge_index, batch, W1, b1, W2, b2, W3, b3, Wg, a_src, a_dst,
           bg, Wc1, bc1, Wc2, bc2):
  src = edge_index[0]
  dst = edge_index[1]
  pad = jnp.arange(EPAD - EE, dtype=I32)
  src3 = jnp.concatenate([src, pad % NN]).reshape(NW, CPW, CH)
  dst3 = jnp.concatenate([dst, NN + pad % (NACC - NN)]).reshape(NW, CPW, CH)

  cntp = _seg_scatter("const", HD, jnp.zeros((8, 128), F32), src3, dst3)
  t1, dinv = _tc1(x, cntp, W1)

  p1 = _seg_scatter("gather", HID, t1, src3, dst3)
  y1, t2 = _tc_layer(p1, t1, dinv, b1.reshape(1, HID), W2, None, HID, HID)
  p2 = _seg_scatter("gather", HID, t2, src3, dst3)
  y2, t3 = _tc_layer(p2, t2, dinv, b2.reshape(1, HID), W3, y1, HID, HID // 2)
  p3 = _seg_scatter("gather", HID // 2, t3, src3, dst3)

  rep = jnp.repeat(jnp.arange(NH), HD)
  asrc = jnp.zeros((NH * HD, NH), F32).at[jnp.arange(NH * HD), rep].set(
      a_src.reshape(-1))
  adst = jnp.zeros((NH * HD, NH), F32).at[jnp.arange(NH * HD), rep].set(
      a_dst.reshape(-1))
  tsrc, tald = _tc_gatprep(p3, t3, dinv, b3.reshape(1, HID // 2),
                           Wg, asrc, adst)

  pgat = _gat_fused(tsrc, tald, src3, dst3)
  return _tc_att_pool(pgat, tsrc, tald, bg.reshape(1, NH * HD),
                      batch.reshape(NN, 1), Wc1, bc1.reshape(1, HID // 2),
                      Wc2, bc2.reshape(1, 2))
